# baseline (speedup 1.0000x reference)
import functools

import jax
import jax.numpy as jnp
from jax.experimental import pallas as pl
from jax.experimental.pallas import tpu as pltpu

_K = 32
_BM = 256

_INT32_MIN = -2147483648


def _sae_block_kernel(x_ref, we_ref, be_ref, wd_ref, bp_ref, o_ref):
    bp = bp_ref[...]
    xc = x_ref[...] - bp
    pre = jax.lax.dot_general(
        xc, we_ref[...], (((1,), (1,)), ((), ())),
        preferred_element_type=jnp.float32,
        precision=jax.lax.Precision.DEFAULT,
    ) + be_ref[...]

    i = jax.lax.bitcast_convert_type(pre, jnp.int32)
    s = i ^ ((i >> 31) & 0x7FFFFFFF)

    ones_cnt = jnp.ones((s.shape[1], 8), dtype=jnp.bfloat16)

    def body(b, tu):
        bit = 31 - b
        cand = tu | (jnp.int32(1) << bit)
        thr = cand ^ _INT32_MIN
        mask_bf = jnp.where(s >= thr, 1.0, 0.0).astype(jnp.bfloat16)
        cnt = jax.lax.dot_general(
            mask_bf, ones_cnt, (((1,), (0,)), ((), ())),
            preferred_element_type=jnp.float32,
            precision=jax.lax.Precision.DEFAULT,
        )[:, :1]
        return jnp.where(cnt >= _K, cand, tu)

    tu0 = jnp.zeros((s.shape[0], 1), jnp.int32)
    tu = jax.lax.fori_loop(0, 32, body, tu0)
    t = tu ^ _INT32_MIN

    z = jnp.where(s >= t, pre, 0.0)
    rec = jax.lax.dot_general(
        z, wd_ref[...], (((1,), (1,)), ((), ())),
        preferred_element_type=jnp.float32,
        precision=jax.lax.Precision.DEFAULT,
    )
    o_ref[...] = rec + bp


def kernel(x, W_enc, b_enc, W_dec, b_pre, model_idx):
    n, d = x.shape
    latent = W_enc.shape[0]
    assert n % _BM == 0
    be2 = b_enc.reshape(1, latent)
    bp2 = b_pre.reshape(1, d)
    return pl.pallas_call(
        _sae_block_kernel,
        grid=(n // _BM,),
        in_specs=[
            pl.BlockSpec((_BM, d), lambda i: (i, 0)),
            pl.BlockSpec((latent, d), lambda i: (0, 0)),
            pl.BlockSpec((1, latent), lambda i: (0, 0)),
            pl.BlockSpec((d, latent), lambda i: (0, 0)),
            pl.BlockSpec((1, d), lambda i: (0, 0)),
        ],
        out_specs=pl.BlockSpec((_BM, d), lambda i: (i, 0)),
        out_shape=jax.ShapeDtypeStruct((n, d), jnp.float32),
        compiler_params=pltpu.CompilerParams(
            dimension_semantics=("parallel",),
        ),
    )(x, W_enc, be2, W_dec, bp2)


# speedup vs baseline: 1.1911x; 1.1911x over previous
import functools

import jax
import jax.numpy as jnp
from jax.experimental import pallas as pl
from jax.experimental.pallas import tpu as pltpu

_K = 32
_BM = 256

_INT32_MIN = -2147483648


def _sae_block_kernel(x_ref, we_ref, be_ref, wd_ref, bp_ref, o_ref):
    bp = bp_ref[...]
    xc = x_ref[...] - bp
    pre = jax.lax.dot_general(
        xc, we_ref[...], (((1,), (1,)), ((), ())),
        preferred_element_type=jnp.float32,
        precision=jax.lax.Precision.DEFAULT,
    ) + be_ref[...]

    bm, lat = pre.shape

    def fwd(f):
        iv = jax.lax.bitcast_convert_type(f, jnp.int32)
        sv = iv ^ ((iv >> 31) & 0x7FFFFFFF)
        return sv ^ _INT32_MIN

    def inv(tuv):
        sv = tuv ^ _INT32_MIN
        iv = sv ^ ((sv >> 31) & 0x7FFFFFFF)
        return jax.lax.bitcast_convert_type(iv, jnp.float32)

    nchunk = 128
    m = pre[:, :nchunk]
    for c in range(1, lat // nchunk):
        m = jnp.maximum(m, pre[:, c * nchunk:(c + 1) * nchunk])

    def mbody(b, tu):
        bit = 31 - b
        cand = tu | (jnp.int32(1) << bit)
        cnt = jnp.sum((m >= inv(cand)).astype(jnp.int32), axis=1, keepdims=True)
        return jnp.where(cnt >= _K, cand, tu)

    lbu = jax.lax.fori_loop(0, 32, mbody, jnp.zeros((bm, 1), jnp.int32))
    ubu = fwd(jnp.max(m, axis=1, keepdims=True))

    diff = lbu ^ ubu
    e = (jax.lax.bitcast_convert_type(
        diff.astype(jnp.float32), jnp.int32) >> 23) & 0xFF
    h = jnp.where(diff < 0, 31, jnp.where(diff == 0, -1, e - 127))
    pm = jnp.where(h >= 31, 0, jnp.int32(-1) << (h + 1))
    tu0 = lbu & pm
    nb = h + 1
    maxnb = jnp.max(nb)

    def body(b, tu):
        bit = nb - 1 - b
        cand = tu | (jnp.int32(1) << jnp.maximum(bit, 0))
        cnt = jnp.sum((pre >= inv(cand)).astype(jnp.int32),
                      axis=1, keepdims=True)
        take = (bit >= 0) & (cnt >= _K)
        return jnp.where(take, cand, tu)

    tu = jax.lax.fori_loop(0, maxnb, body, tu0)

    z = jnp.where(pre >= inv(tu), pre, 0.0)
    rec = jax.lax.dot_general(
        z, wd_ref[...], (((1,), (1,)), ((), ())),
        preferred_element_type=jnp.float32,
        precision=jax.lax.Precision.DEFAULT,
    )
    o_ref[...] = rec + bp


def kernel(x, W_enc, b_enc, W_dec, b_pre, model_idx):
    n, d = x.shape
    latent = W_enc.shape[0]
    assert n % _BM == 0
    be2 = b_enc.reshape(1, latent)
    bp2 = b_pre.reshape(1, d)
    return pl.pallas_call(
        _sae_block_kernel,
        grid=(n // _BM,),
        in_specs=[
            pl.BlockSpec((_BM, d), lambda i: (i, 0)),
            pl.BlockSpec((latent, d), lambda i: (0, 0)),
            pl.BlockSpec((1, latent), lambda i: (0, 0)),
            pl.BlockSpec((d, latent), lambda i: (0, 0)),
            pl.BlockSpec((1, d), lambda i: (0, 0)),
        ],
        out_specs=pl.BlockSpec((_BM, d), lambda i: (i, 0)),
        out_shape=jax.ShapeDtypeStruct((n, d), jnp.float32),
        compiler_params=pltpu.CompilerParams(
            dimension_semantics=("parallel",),
        ),
    )(x, W_enc, be2, W_dec, bp2)


# speedup vs baseline: 1.7020x; 1.4289x over previous
import functools

import jax
import jax.numpy as jnp
from jax.experimental import pallas as pl
from jax.experimental.pallas import tpu as pltpu

_K = 32
_BM = 256

_INT32_MIN = -2147483648


def _sae_block_kernel(x_ref, we_ref, be_ref, wd_ref, bp_ref, o_ref):
    bp = bp_ref[...]
    xc = x_ref[...] - bp
    pre = jax.lax.dot_general(
        xc, we_ref[...], (((1,), (1,)), ((), ())),
        preferred_element_type=jnp.float32,
        precision=jax.lax.Precision.DEFAULT,
    ) + be_ref[...]

    bm, lat = pre.shape

    def fwd(f):
        iv = jax.lax.bitcast_convert_type(f, jnp.int32)
        sv = iv ^ ((iv >> 31) & 0x7FFFFFFF)
        return sv ^ _INT32_MIN

    def inv(tuv):
        sv = tuv ^ _INT32_MIN
        iv = sv ^ ((sv >> 31) & 0x7FFFFFFF)
        return jax.lax.bitcast_convert_type(iv, jnp.float32)

    nchunk = 128
    m = pre[:, :nchunk]
    for c in range(1, lat // nchunk):
        m = jnp.maximum(m, pre[:, c * nchunk:(c + 1) * nchunk])

    lo0 = fwd(jnp.min(m, axis=1, keepdims=True))
    hi0 = fwd(jnp.max(m, axis=1, keepdims=True)) + 1

    def cond(st):
        b, lo, hi, done, tau = st
        return (b < 40) & (jnp.min(done) == 0)

    def bodyw(st):
        b, lo, hi, done, tau = st
        mid = lo + jax.lax.shift_right_logical(hi - lo, 1)
        cnt = jnp.sum((pre >= inv(mid)).astype(jnp.int32),
                      axis=1, keepdims=True)
        notdone = done == 0
        exact = (cnt == _K) & notdone
        stuck = (hi - lo <= 1) & notdone
        tau = jnp.where(exact, mid, jnp.where(stuck, lo, tau))
        ndone_b = exact | stuck
        ge = cnt >= _K
        keep = ndone_b | (done != 0)
        nlo = jnp.where(keep | jnp.logical_not(ge), lo, mid)
        nhi = jnp.where(keep | ge, hi, mid)
        ndone = jnp.where(ndone_b, jnp.int32(1), done)
        return (b + 1, nlo, nhi, ndone, tau)

    st0 = (jnp.int32(0), lo0, hi0,
           jnp.zeros((bm, 1), jnp.int32), lo0)
    _, _, _, _, tau = jax.lax.while_loop(cond, bodyw, st0)

    z = jnp.where(pre >= inv(tau), pre, 0.0)
    rec = jax.lax.dot_general(
        z, wd_ref[...], (((1,), (1,)), ((), ())),
        preferred_element_type=jnp.float32,
        precision=jax.lax.Precision.DEFAULT,
    )
    o_ref[...] = rec + bp


def kernel(x, W_enc, b_enc, W_dec, b_pre, model_idx):
    n, d = x.shape
    latent = W_enc.shape[0]
    assert n % _BM == 0
    be2 = b_enc.reshape(1, latent)
    bp2 = b_pre.reshape(1, d)
    return pl.pallas_call(
        _sae_block_kernel,
        grid=(n // _BM,),
        in_specs=[
            pl.BlockSpec((_BM, d), lambda i: (i, 0)),
            pl.BlockSpec((latent, d), lambda i: (0, 0)),
            pl.BlockSpec((1, latent), lambda i: (0, 0)),
            pl.BlockSpec((d, latent), lambda i: (0, 0)),
            pl.BlockSpec((1, d), lambda i: (0, 0)),
        ],
        out_specs=pl.BlockSpec((_BM, d), lambda i: (i, 0)),
        out_shape=jax.ShapeDtypeStruct((n, d), jnp.float32),
        compiler_params=pltpu.CompilerParams(
            dimension_semantics=("parallel",),
        ),
    )(x, W_enc, be2, W_dec, bp2)
